# Initial kernel scaffold; baseline (speedup 1.0000x reference)
import jax
import jax.numpy as jnp
from jax.experimental import pallas as pl


def kernel(x, W_start, b_start, gate_w, W1, b1, W2, b2, W_proj, b_proj):
    raise NotImplementedError("write your pallas kernel here")


# speedup vs baseline: 1.5525x; 1.5525x over previous
import jax
import jax.numpy as jnp
from jax.experimental import pallas as pl
from jax.experimental.pallas import tpu as pltpu

_B, _S, _N = 16, 336, 64
_D, _F, _E, _K, _L, _P = 16, 64, 4, 2, 2, 96
_ZF = _L * _E * _F
_AC = _L * _K * _F
_SL = 14
_R = _SL * _N
_T = _S // _SL
_EPS = 1e-5
_GELU_C = 0.7978845608028654


def _moe_kernel(x3_ref, xf_ref, w1t_ref, b1c_ref, w2r_ref, b2t_ref,
                wst_ref, bst_ref, idx_ref, sm_ref, q_ref):
    b = pl.program_id(0)
    xb = x3_ref[0]
    mean_r = jnp.mean(xb, axis=0, keepdims=True)
    m2_r = jnp.mean(xb * xb, axis=0, keepdims=True)
    std_r = jnp.sqrt(m2_r - mean_r * mean_r + _EPS)
    smean = jnp.sum(std_r) * (1.0 / _N)
    meanrow = jnp.concatenate([mean_r] * _SL, axis=1)
    stdrow = jnp.concatenate([std_r] * _SL, axis=1)

    les = []
    gs = []
    for l in range(_L):
        for k in range(_K):
            les.append(l * _E + idx_ref[l, b, k])
            gs.append(sm_ref[l, b, k])

    r_i = jax.lax.broadcasted_iota(jnp.int32, (_AC, _ZF), 0)
    c_i = jax.lax.broadcasted_iota(jnp.int32, (_AC, _ZF), 1)
    rk = r_i // _F
    le_row = ((rk == 0) * les[0] + (rk == 1) * les[1]
              + (rk == 2) * les[2] + (rk == 3) * les[3])
    sel_u = (((c_i // _F) == le_row)
             & ((c_i % _F) == (r_i % _F))).astype(jnp.float32)

    r2 = jax.lax.broadcasted_iota(jnp.int32, (_ZF, _AC), 0)
    c2 = jax.lax.broadcasted_iota(jnp.int32, (_ZF, _AC), 1)
    ck = c2 // _F
    le_col = ((ck == 0) * les[0] + (ck == 1) * les[1]
              + (ck == 2) * les[2] + (ck == 3) * les[3])
    g_col = ((ck == 0) * gs[0] + (ck == 1) * gs[1]
             + (ck == 2) * gs[2] + (ck == 3) * gs[3])
    sel_g = (((r2 // _F) == le_col)
             & ((r2 % _F) == (c2 % _F))).astype(jnp.float32) * g_col

    u_col = jnp.dot(w1t_ref[...], wst_ref[...],
                    preferred_element_type=jnp.float32)
    c_col = jnp.dot(w1t_ref[...], bst_ref[...],
                    preferred_element_type=jnp.float32) + b1c_ref[...]
    ua = jnp.dot(sel_u, u_col, preferred_element_type=jnp.float32)
    ca = jnp.dot(sel_u, c_col, preferred_element_type=jnp.float32)
    w2a = jnp.dot(w2r_ref[...], sel_g,
                  preferred_element_type=jnp.float32)

    i8 = jax.lax.broadcasted_iota(jnp.int32, (_L * _E, 1), 0)
    sel8 = ((i8 == les[0]) * gs[0] + (i8 == les[1]) * gs[1]
            + (i8 == les[2]) * gs[2] + (i8 == les[3]) * gs[3])
    b2g = jnp.dot(b2t_ref[...], sel8.astype(jnp.float32),
                  preferred_element_type=jnp.float32)
    c_const = 2.0 * smean * bst_ref[...] + smean * b2g

    sr = jax.lax.broadcasted_iota(jnp.int32, (_R, _SL), 0)
    sc = jax.lax.broadcasted_iota(jnp.int32, (_R, _SL), 1)
    segm = ((sr // _N) == sc).astype(jnp.float32)
    wst = wst_ref[...]

    def body(t, carry):
        off = pl.multiple_of(t * _R, 128)
        xr = xf_ref[0, 0:1, pl.ds(off, _R)]
        xc = xr - meanrow
        v = xc / stdrow
        act = ua * v + ca
        a3 = act * act * act
        gl = 0.5 * act * (1.0 + jnp.tanh(_GELU_C * (act + 0.044715 * a3)))
        ys = jnp.dot(w2a, gl, preferred_element_type=jnp.float32)
        ysc = ys * stdrow
        q = jnp.dot(ysc, segm, preferred_element_type=jnp.float32)
        wxn = jnp.dot(xc, segm, preferred_element_type=jnp.float32)
        qf = q * (1.0 / _N) + (2.0 / _N) * (wst * wxn) + c_const
        q_ref[0, t] = qf
        return carry

    jax.lax.fori_loop(0, _T, body, 0)


def _proj_kernel(x3_ref, td_ref, wp_ref, bp_ref, o_ref):
    x = x3_ref[...]
    mean = jnp.mean(x, axis=1)
    m2 = jnp.mean(x * x, axis=1)
    std = jnp.sqrt(m2 - mean * mean + _EPS)
    smean = jnp.mean(std, axis=1, keepdims=True)
    mmean = jnp.mean(mean, axis=1, keepdims=True)
    o = jnp.dot(td_ref[...], wp_ref[...],
                preferred_element_type=jnp.float32)
    o_ref[...] = o + smean * bp_ref[...] + mmean


def kernel(x, W_start, b_start, gate_w, W1, b1, W2, b2, W_proj, b_proj):
    f32 = jnp.float32
    mean = jnp.mean(x, axis=1, keepdims=True)
    std = jnp.sqrt(jnp.var(x, axis=1, keepdims=True) + _EPS)
    xn = (x - mean) / std
    h0 = xn[..., None] @ W_start + b_start
    pooled = h0.mean(axis=(1, 2))
    idxs, sms = [], []
    for l in range(_L):
        logits = pooled @ gate_w[l]
        vals, idx = jax.lax.top_k(logits, _K)
        sms.append(jax.nn.softmax(vals, axis=-1))
        idxs.append(idx)
    idx_arr = jnp.stack(idxs).astype(jnp.int32)
    sm_arr = jnp.stack(sms).astype(f32)

    x_flat = x.reshape(_B, 1, _S * _N)
    w1t = W1.reshape(_L * _E, _D, _F).transpose(0, 2, 1).reshape(_ZF, _D)
    b1c = b1.reshape(_ZF, 1)
    w2r = W2.reshape(_L * _E, _F, _D).transpose(2, 0, 1).reshape(_D, _ZF)
    b2t = b2.reshape(_L * _E, _D).T
    wstT = W_start.reshape(_D, 1)
    bstT = b_start.reshape(_D, 1)
    wp_r = W_proj.reshape(_S, _D, _P).transpose(1, 0, 2).reshape(_S * _D, _P)
    bp_row = b_proj.reshape(1, _P)

    q = pl.pallas_call(
        _moe_kernel,
        grid=(_B,),
        in_specs=[
            pl.BlockSpec((1, _S, _N), lambda b: (b, 0, 0)),
            pl.BlockSpec((1, 1, _S * _N), lambda b: (b, 0, 0)),
            pl.BlockSpec((_ZF, _D), lambda b: (0, 0)),
            pl.BlockSpec((_ZF, 1), lambda b: (0, 0)),
            pl.BlockSpec((_D, _ZF), lambda b: (0, 0)),
            pl.BlockSpec((_D, _L * _E), lambda b: (0, 0)),
            pl.BlockSpec((_D, 1), lambda b: (0, 0)),
            pl.BlockSpec((_D, 1), lambda b: (0, 0)),
            pl.BlockSpec(memory_space=pltpu.SMEM),
            pl.BlockSpec(memory_space=pltpu.SMEM),
        ],
        out_specs=pl.BlockSpec((1, _T, _D, _SL), lambda b: (b, 0, 0, 0)),
        out_shape=jax.ShapeDtypeStruct((_B, _T, _D, _SL), f32),
    )(x.astype(f32), x_flat, w1t, b1c, w2r, b2t, wstT, bstT, idx_arr, sm_arr)

    td = q.transpose(0, 2, 1, 3).reshape(_B, _S * _D)

    out = pl.pallas_call(
        _proj_kernel,
        out_shape=jax.ShapeDtypeStruct((_B, _P), f32),
    )(x.astype(f32), td, wp_r, bp_row)
    return out


# speedup vs baseline: 2.1160x; 1.3630x over previous
import jax
import jax.numpy as jnp
from jax.experimental import pallas as pl
from jax.experimental.pallas import tpu as pltpu

_B, _S, _N = 16, 336, 64
_D, _F, _E, _K, _L, _P = 16, 64, 4, 2, 2, 96
_ZF = _L * _E * _F
_AC = _L * _K * _F
_SL = 14
_R = _SL * _N
_T = _S // _SL
_EPS = 1e-5
_GELU_C = 0.7978845608028654


def _moe_kernel(x3_ref, xf_ref, w1t_ref, b1c_ref, w2r_ref, b2t_ref,
                wst_ref, bst_ref, idx_ref, sm_ref, q_ref):
    b = pl.program_id(0)
    xb = x3_ref[0]
    mean_r = jnp.mean(xb, axis=0, keepdims=True)
    m2_r = jnp.mean(xb * xb, axis=0, keepdims=True)
    std_r = jnp.sqrt(m2_r - mean_r * mean_r + _EPS)
    smean = jnp.sum(std_r) * (1.0 / _N)
    meanrow = jnp.concatenate([mean_r] * _SL, axis=1)
    stdrow = jnp.concatenate([std_r] * _SL, axis=1)

    les = []
    gs = []
    for l in range(_L):
        for k in range(_K):
            les.append(l * _E + idx_ref[l, b, k])
            gs.append(sm_ref[l, b, k])

    r_i = jax.lax.broadcasted_iota(jnp.int32, (_AC, _ZF), 0)
    c_i = jax.lax.broadcasted_iota(jnp.int32, (_AC, _ZF), 1)
    rk = r_i // _F
    le_row = ((rk == 0) * les[0] + (rk == 1) * les[1]
              + (rk == 2) * les[2] + (rk == 3) * les[3])
    sel_u = (((c_i // _F) == le_row)
             & ((c_i % _F) == (r_i % _F))).astype(jnp.float32)

    r2 = jax.lax.broadcasted_iota(jnp.int32, (_ZF, _AC), 0)
    c2 = jax.lax.broadcasted_iota(jnp.int32, (_ZF, _AC), 1)
    ck = c2 // _F
    le_col = ((ck == 0) * les[0] + (ck == 1) * les[1]
              + (ck == 2) * les[2] + (ck == 3) * les[3])
    g_col = ((ck == 0) * gs[0] + (ck == 1) * gs[1]
             + (ck == 2) * gs[2] + (ck == 3) * gs[3])
    sel_g = (((r2 // _F) == le_col)
             & ((r2 % _F) == (c2 % _F))).astype(jnp.float32) * g_col

    u_col = jnp.dot(w1t_ref[...], wst_ref[...],
                    preferred_element_type=jnp.float32)
    c_col = jnp.dot(w1t_ref[...], bst_ref[...],
                    preferred_element_type=jnp.float32) + b1c_ref[...]
    ua = jnp.dot(sel_u, u_col, preferred_element_type=jnp.float32)
    ca = jnp.dot(sel_u, c_col, preferred_element_type=jnp.float32)
    w2a = jnp.dot(w2r_ref[...], sel_g,
                  preferred_element_type=jnp.float32)

    i8 = jax.lax.broadcasted_iota(jnp.int32, (_L * _E, 1), 0)
    sel8 = ((i8 == les[0]) * gs[0] + (i8 == les[1]) * gs[1]
            + (i8 == les[2]) * gs[2] + (i8 == les[3]) * gs[3])
    b2g = jnp.dot(b2t_ref[...], sel8.astype(jnp.float32),
                  preferred_element_type=jnp.float32)
    c_const = 2.0 * smean * bst_ref[...] + smean * b2g

    sr = jax.lax.broadcasted_iota(jnp.int32, (_R, _SL), 0)
    sc = jax.lax.broadcasted_iota(jnp.int32, (_R, _SL), 1)
    segm = ((sr // _N) == sc).astype(jnp.float32)
    wst = wst_ref[...]

    bf16 = jnp.bfloat16
    ua_h = ua.astype(bf16)
    ca_h = ca.astype(bf16)
    w2a_h = (w2a * (0.5 / _N)).astype(bf16)

    def body(t, carry):
        off = pl.multiple_of(t * _R, 128)
        xr = xf_ref[0, 0:1, pl.ds(off, _R)]
        xc = xr - meanrow
        v = (xc / stdrow).astype(bf16)
        act = ua_h * v + ca_h
        a3 = act * act * act
        gl = act * (1.0 + jnp.tanh(_GELU_C * (act + 0.044715 * a3)))
        ys = jnp.dot(w2a_h, gl, preferred_element_type=jnp.float32)
        ysc = ys * stdrow
        q = jnp.dot(ysc, segm, preferred_element_type=jnp.float32)
        wxn = jnp.dot(xc, segm, preferred_element_type=jnp.float32)
        qf = q + (2.0 / _N) * (wst * wxn) + c_const
        q_ref[0, t] = qf
        return carry

    jax.lax.fori_loop(0, _T, body, 0)


def _proj_kernel(x3_ref, td_ref, wp_ref, bp_ref, o_ref):
    x = x3_ref[...]
    mean = jnp.mean(x, axis=1)
    m2 = jnp.mean(x * x, axis=1)
    std = jnp.sqrt(m2 - mean * mean + _EPS)
    smean = jnp.mean(std, axis=1, keepdims=True)
    mmean = jnp.mean(mean, axis=1, keepdims=True)
    o = jnp.dot(td_ref[...], wp_ref[...],
                preferred_element_type=jnp.float32)
    o_ref[...] = o + smean * bp_ref[...] + mmean


def kernel(x, W_start, b_start, gate_w, W1, b1, W2, b2, W_proj, b_proj):
    f32 = jnp.float32
    mean = jnp.mean(x, axis=1, keepdims=True)
    std = jnp.sqrt(jnp.var(x, axis=1, keepdims=True) + _EPS)
    xn = (x - mean) / std
    h0 = xn[..., None] @ W_start + b_start
    pooled = h0.mean(axis=(1, 2))
    idxs, sms = [], []
    for l in range(_L):
        logits = pooled @ gate_w[l]
        vals, idx = jax.lax.top_k(logits, _K)
        sms.append(jax.nn.softmax(vals, axis=-1))
        idxs.append(idx)
    idx_arr = jnp.stack(idxs).astype(jnp.int32)
    sm_arr = jnp.stack(sms).astype(f32)

    x_flat = x.reshape(_B, 1, _S * _N)
    w1t = W1.reshape(_L * _E, _D, _F).transpose(0, 2, 1).reshape(_ZF, _D)
    b1c = b1.reshape(_ZF, 1)
    w2r = W2.reshape(_L * _E, _F, _D).transpose(2, 0, 1).reshape(_D, _ZF)
    b2t = b2.reshape(_L * _E, _D).T
    wstT = W_start.reshape(_D, 1)
    bstT = b_start.reshape(_D, 1)
    wp_r = W_proj.reshape(_S, _D, _P).transpose(1, 0, 2).reshape(_S * _D, _P)
    bp_row = b_proj.reshape(1, _P)

    q = pl.pallas_call(
        _moe_kernel,
        grid=(_B,),
        in_specs=[
            pl.BlockSpec((1, _S, _N), lambda b: (b, 0, 0)),
            pl.BlockSpec((1, 1, _S * _N), lambda b: (b, 0, 0)),
            pl.BlockSpec((_ZF, _D), lambda b: (0, 0)),
            pl.BlockSpec((_ZF, 1), lambda b: (0, 0)),
            pl.BlockSpec((_D, _ZF), lambda b: (0, 0)),
            pl.BlockSpec((_D, _L * _E), lambda b: (0, 0)),
            pl.BlockSpec((_D, 1), lambda b: (0, 0)),
            pl.BlockSpec((_D, 1), lambda b: (0, 0)),
            pl.BlockSpec(memory_space=pltpu.SMEM),
            pl.BlockSpec(memory_space=pltpu.SMEM),
        ],
        out_specs=pl.BlockSpec((1, _T, _D, _SL), lambda b: (b, 0, 0, 0)),
        out_shape=jax.ShapeDtypeStruct((_B, _T, _D, _SL), f32),
    )(x.astype(f32), x_flat, w1t, b1c, w2r, b2t, wstT, bstT, idx_arr, sm_arr)

    td = q.transpose(0, 2, 1, 3).reshape(_B, _S * _D)

    out = pl.pallas_call(
        _proj_kernel,
        out_shape=jax.ShapeDtypeStruct((_B, _P), f32),
    )(x.astype(f32), td, wp_r, bp_row)
    return out
